# baseline (speedup 1.0000x reference)
import functools

import jax
import jax.numpy as jnp
from jax import lax
from jax.experimental import pallas as pl
from jax.experimental.pallas import tpu as pltpu
from jax.experimental.pallas import tpu_sc as plsc

N = 10000
E = 160000
DF = 128
DE = 16
H = 32
B = 500
BP = 512
MP_STEPS = 3
S2S_STEPS = 3

NC = 2
NS = 16
NW = NC * NS
EPAD = 163840
CHUNK = 1024
GC = 256
NCHUNK = EPAD // (NW * GC)
IB = 128
JPER = GC // IB
W = 128
HALF = N // 2
ACCROWS = 5120
TPT = EPAD // NS
SCCH = TPT // GC

@functools.cache
def _sc_kernels():
    mesh = plsc.VectorSubcoreMesh(core_axis_name="c", subcore_axis_name="s",
                                  num_cores=NC, num_subcores=NS)

    @functools.partial(
        pl.kernel,
        out_type=jax.ShapeDtypeStruct((EPAD, W), jnp.float32),
        mesh=mesh,
        scratch_types=[
            pltpu.VMEM((JPER, IB), jnp.int32),
            pltpu.VMEM((JPER, IB), jnp.int32),
            pltpu.VMEM((GC, W), jnp.float32),
            pltpu.VMEM((GC, W), jnp.float32),
            pltpu.SemaphoreType.DMA,
            pltpu.SemaphoreType.DMA,
        ],
    )
    def sc_gather(table_hbm, idx2_hbm, out_hbm, ix0, ix1, rb0, rb1,
                  sem_g, sem_wb):
        ixb = (ix0, ix1)
        rbb = (rb0, rb1)
        wid = lax.axis_index("s") * NC + lax.axis_index("c")
        wbase = pl.multiple_of(wid * (NCHUNK * GC), GC)
        wrow = pl.multiple_of(wid * (NCHUNK * JPER), JPER)
        pltpu.sync_copy(idx2_hbm.at[pl.ds(wrow, JPER)], ixb[0])
        wb = [None] * NCHUNK
        for c in range(NCHUNK):
            b = c % 2
            if c >= 2:
                wb[c - 2].wait()
            g = [pltpu.async_copy(table_hbm.at[ixb[b].at[j]],
                                  rbb[b].at[pl.ds(j * IB, IB)], sem_g)
                 for j in range(JPER)]
            if c + 1 < NCHUNK:
                pltpu.sync_copy(
                    idx2_hbm.at[pl.ds(wrow + (c + 1) * JPER, JPER)],
                    ixb[(c + 1) % 2])
            for d in g:
                d.wait()
            wb[c] = pltpu.async_copy(
                rbb[b], out_hbm.at[pl.ds(wbase + c * GC, GC)], sem_wb)
        wb[NCHUNK - 2].wait()
        wb[NCHUNK - 1].wait()

    @functools.partial(
        pl.kernel,
        out_type=jax.ShapeDtypeStruct((NC, ACCROWS, W), jnp.float32),
        mesh=mesh,
        scratch_types=[
            pltpu.VMEM((JPER, IB), jnp.int32),
            pltpu.VMEM((JPER, IB), jnp.int32),
            pltpu.VMEM((GC, W), jnp.float32),
            pltpu.VMEM((GC, W), jnp.float32),
            pltpu.VMEM_SHARED((ACCROWS, W), jnp.float32),
            pltpu.SemaphoreType.DMA,
            pltpu.SemaphoreType.DMA,
            pltpu.SemaphoreType.DMA,
        ],
    )
    def sc_scatter(msg_hbm, dst2_hbm, z_hbm, out_hbm, ix0, ix1, bf0, bf1,
                   acc_s, sem_ix, sem_in, sem_sc):
        ixb = (ix0, ix1)
        bfb = (bf0, bf1)
        cid = lax.axis_index("c")
        sid = lax.axis_index("s")

        @pl.when(sid == 0)
        def _init():
            pltpu.sync_copy(z_hbm, acc_s)

        nbase = cid * HALF
        tbase = pl.multiple_of(sid * TPT, GC)
        trow = pl.multiple_of(sid * (TPT // IB), JPER)
        plsc.subcore_barrier()

        def stage(c):
            b = c % 2
            di = pltpu.async_copy(
                dst2_hbm.at[pl.ds(trow + c * JPER, JPER)], ixb[b], sem_ix)
            dm = pltpu.async_copy(
                msg_hbm.at[pl.ds(tbase + c * GC, GC)], bfb[b], sem_in)
            return di, dm

        st = {0: stage(0), 1: stage(1)}
        sc = {}
        for c in range(SCCH):
            b = c % 2
            di, dm = st[c]
            di.wait()
            dm.wait()
            for r in range(JPER):
                for kk in range(IB // 16):
                    v = ixb[b][r, pl.ds(kk * 16, 16)] - nbase
                    ok = (v >= 0) & (v < HALF)
                    ixb[b][r, pl.ds(kk * 16, 16)] = jnp.where(ok, v, HALF)
            sc[c] = [pltpu.async_copy(bfb[b].at[pl.ds(r * IB, IB)],
                                      acc_s.at[ixb[b].at[r]], sem_sc,
                                      add=True)
                     for r in range(JPER)]
            if c + 2 < SCCH:
                for d in sc[c]:
                    d.wait()
                st[c + 2] = stage(c + 2)
        for d in sc[SCCH - 2] + sc[SCCH - 1]:
            d.wait()
        plsc.subcore_barrier()

        @pl.when(sid == 0)
        def _flush():
            pltpu.sync_copy(acc_s, out_hbm.at[cid])

    return sc_gather, sc_scatter


def _sc_gather(table, idx2):
    return _sc_kernels()[0](table, idx2)


def _sc_scatter(msg, dst2, z):
    return _sc_kernels()[1](msg, dst2, z)


def _init_body(x_ref, w_ref, b_ref, o_ref):
    o = jax.nn.relu(
        jnp.dot(x_ref[...], w_ref[...], preferred_element_type=jnp.float32, precision=lax.Precision.HIGHEST)
        + b_ref[...])
    o_ref[...] = jnp.concatenate(
        [o, jnp.zeros((N, W - H), jnp.float32)], axis=1)


def _tc_init(x, W0, b0):
    return pl.pallas_call(
        _init_body,
        out_shape=jax.ShapeDtypeStruct((N, W), jnp.float32),
    )(x, W0, b0.reshape(1, H))


def _msg_body(eat_ref, os_ref, we1_ref, be1_ref, we2_ref, be2_ref,
              r_ref, s_ref, o_ref):
    i = pl.program_id(0)
    hid = jax.nn.relu(
        lax.dot_general(eat_ref[...], we1_ref[...], (((0,), (0,)), ((), ())),
                        preferred_element_type=jnp.float32)
        + be1_ref[...])
    ew = jnp.dot(hid, we2_ref[...], preferred_element_type=jnp.float32) \
        + be2_ref[...]
    osx = jnp.dot(os_ref[:, :H], r_ref[...],
                  preferred_element_type=jnp.float32,
                  precision=lax.Precision.HIGHEST)
    msg = jnp.dot(ew * osx, s_ref[...],
                  preferred_element_type=jnp.float32,
                  precision=lax.Precision.HIGHEST)
    gid = i * CHUNK + lax.broadcasted_iota(jnp.int32, (CHUNK, 1), 0)
    real = jnp.where(gid < E, 1.0, 0.0)
    msg = jnp.where(gid < E, msg, 0.0)
    o_ref[...] = jnp.concatenate(
        [msg, real, jnp.zeros((CHUNK, W - H - 1), jnp.float32)], axis=1)


def _tc_msg(eat, osrc, We1, be1, We2, be2, Rm, Sm):
    nblk = EPAD // CHUNK
    return pl.pallas_call(
        _msg_body,
        grid=(nblk,),
        in_specs=[
            pl.BlockSpec((DE, CHUNK), lambda i: (0, i)),
            pl.BlockSpec((CHUNK, W), lambda i: (i, 0)),
            pl.BlockSpec((DE, 2 * H), lambda i: (0, 0)),
            pl.BlockSpec((1, 2 * H), lambda i: (0, 0)),
            pl.BlockSpec((2 * H, H * H), lambda i: (0, 0)),
            pl.BlockSpec((1, H * H), lambda i: (0, 0)),
            pl.BlockSpec((H, H * H), lambda i: (0, 0)),
            pl.BlockSpec((H * H, H), lambda i: (0, 0)),
        ],
        out_specs=pl.BlockSpec((CHUNK, W), lambda i: (i, 0)),
        out_shape=jax.ShapeDtypeStruct((EPAD, W), jnp.float32),
    )(eat, osrc, We1, be1.reshape(1, 2 * H), We2, be2.reshape(1, H * H),
      Rm, Sm)


def _update_body(out_ref, h_ref, p_ref, wroot_ref, broot_ref,
                 wi_ref, bi_ref, wh_ref, bh_ref, o_ref):
    out = out_ref[:, :H]
    h = h_ref[:, :H]
    psum = jnp.concatenate(
        [p_ref[0, :HALF, :], p_ref[1, :HALF, :]], axis=0)
    deg = jnp.maximum(psum[:, H:H + 1], 1.0)
    aggr = psum[:, :H] / deg
    m = jax.nn.relu(
        jnp.dot(out, wroot_ref[...], preferred_element_type=jnp.float32, precision=lax.Precision.HIGHEST)
        + broot_ref[...] + aggr)
    gi = jnp.dot(m, wi_ref[...], preferred_element_type=jnp.float32, precision=lax.Precision.HIGHEST) \
        + bi_ref[...]
    gh = jnp.dot(h, wh_ref[...], preferred_element_type=jnp.float32, precision=lax.Precision.HIGHEST) \
        + bh_ref[...]
    ir, iz, inn = gi[:, :H], gi[:, H:2 * H], gi[:, 2 * H:]
    hr, hz, hn = gh[:, :H], gh[:, H:2 * H], gh[:, 2 * H:]
    r = jax.nn.sigmoid(ir + hr)
    z = jax.nn.sigmoid(iz + hz)
    n = jnp.tanh(inn + r * hn)
    hnew = (1.0 - z) * n + z * h
    o_ref[...] = jnp.concatenate(
        [hnew, jnp.zeros((N, W - H), jnp.float32)], axis=1)


def _tc_update(out, h, p, Wroot, broot, Wi, bi, Wh, bh):
    return pl.pallas_call(
        _update_body,
        out_shape=jax.ShapeDtypeStruct((N, W), jnp.float32),
    )(out, h, p, Wroot, broot.reshape(1, H), Wi, bi.reshape(1, 3 * H),
      Wh, bh.reshape(1, 3 * H))


def _lstm_body(qp_ref, racc_ref, den_ref, hl_ref, cl_ref,
               wli_ref, bli_ref, wlh_ref, blh_ref, hlo_ref, clo_ref):
    r = racc_ref[...] / jnp.maximum(den_ref[...], 1e-16)
    q_star = jnp.concatenate([qp_ref[...], r], axis=1)
    g = jnp.dot(q_star, wli_ref[...], preferred_element_type=jnp.float32, precision=lax.Precision.HIGHEST) \
        + bli_ref[...] \
        + jnp.dot(hl_ref[...], wlh_ref[...],
                  preferred_element_type=jnp.float32, precision=lax.Precision.HIGHEST) + blh_ref[...]
    ig = jax.nn.sigmoid(g[:, :H])
    fg = jax.nn.sigmoid(g[:, H:2 * H])
    gg = jnp.tanh(g[:, 2 * H:3 * H])
    og = jax.nn.sigmoid(g[:, 3 * H:])
    cl = fg * cl_ref[...] + ig * gg
    clo_ref[...] = cl
    hlo_ref[...] = og * jnp.tanh(cl)


def _tc_lstm(qp, racc, den, hl, cl, Wli, bli, Wlh, blh):
    return pl.pallas_call(
        _lstm_body,
        out_shape=(
            jax.ShapeDtypeStruct((BP, H), jnp.float32),
            jax.ShapeDtypeStruct((BP, H), jnp.float32),
        ),
    )(qp, racc, den, hl, cl, Wli, bli.reshape(1, 4 * H),
      Wlh, blh.reshape(1, 4 * H))


NB = 2000
NBLK = N // NB


def _passA_body(out_ref, q_ref, b_ref, e_ref, gmax_ref):
    i = pl.program_id(0)
    oh = jnp.where(
        lax.broadcasted_iota(jnp.int32, (NB, BP), 1) == b_ref[...],
        1.0, 0.0)
    qn = jnp.dot(oh, q_ref[...], preferred_element_type=jnp.float32, precision=lax.Precision.HIGHEST)
    e = jnp.sum(out_ref[:, :H] * qn, axis=1, keepdims=True)
    e_ref[...] = e

    @pl.when(i == 0)
    def _():
        gmax_ref[...] = jnp.full((1, 1), -3e38, jnp.float32)

    gmax_ref[...] = jnp.maximum(gmax_ref[...], jnp.max(e))


def _tc_passA(out, q, batch2):
    return pl.pallas_call(
        _passA_body,
        grid=(NBLK,),
        in_specs=[
            pl.BlockSpec((NB, W), lambda i: (i, 0)),
            pl.BlockSpec((BP, H), lambda i: (0, 0)),
            pl.BlockSpec((NB, 1), lambda i: (i, 0)),
        ],
        out_specs=(
            pl.BlockSpec((NB, 1), lambda i: (i, 0)),
            pl.BlockSpec((1, 1), lambda i: (0, 0)),
        ),
        out_shape=(
            jax.ShapeDtypeStruct((N, 1), jnp.float32),
            jax.ShapeDtypeStruct((1, 1), jnp.float32),
        ),
    )(out, q, batch2)


def _passB_body(e_ref, gmax_ref, out_ref, b_ref, den_ref, racc_ref):
    i = pl.program_id(0)
    a = jnp.exp(e_ref[...] - gmax_ref[0, 0])
    oh = jnp.where(
        lax.broadcasted_iota(jnp.int32, (NB, BP), 1) == b_ref[...],
        1.0, 0.0)
    dn = lax.dot_general(oh, a, (((0,), (0,)), ((), ())),
                         preferred_element_type=jnp.float32, precision=lax.Precision.HIGHEST)
    rc = lax.dot_general(oh, a * out_ref[:, :H], (((0,), (0,)), ((), ())),
                         preferred_element_type=jnp.float32, precision=lax.Precision.HIGHEST)

    @pl.when(i == 0)
    def _():
        den_ref[...] = jnp.zeros((BP, 1), jnp.float32)
        racc_ref[...] = jnp.zeros((BP, H), jnp.float32)

    den_ref[...] += dn
    racc_ref[...] += rc


def _tc_passB(e, gmax, out, batch2):
    return pl.pallas_call(
        _passB_body,
        grid=(NBLK,),
        in_specs=[
            pl.BlockSpec((NB, 1), lambda i: (i, 0)),
            pl.BlockSpec((1, 1), lambda i: (0, 0)),
            pl.BlockSpec((NB, W), lambda i: (i, 0)),
            pl.BlockSpec((NB, 1), lambda i: (i, 0)),
        ],
        out_specs=(
            pl.BlockSpec((BP, 1), lambda i: (0, 0)),
            pl.BlockSpec((BP, H), lambda i: (0, 0)),
        ),
        out_shape=(
            jax.ShapeDtypeStruct((BP, 1), jnp.float32),
            jax.ShapeDtypeStruct((BP, H), jnp.float32),
        ),
    )(e, gmax, out, batch2)


def _head_body(q_ref, racc_ref, den_ref, wr1_ref, br1_ref, g_ref, be_ref,
               wr2_ref, br2_ref, y_ref):
    r = racc_ref[...] / jnp.maximum(den_ref[...], 1e-16)
    q_star = jnp.concatenate([q_ref[...], r], axis=1)
    z1 = jnp.dot(q_star, wr1_ref[...], preferred_element_type=jnp.float32, precision=lax.Precision.HIGHEST) \
        + br1_ref[...]
    mask = jnp.where(
        lax.broadcasted_iota(jnp.int32, (BP, 1), 0) < B, 1.0, 0.0)
    zm = z1 * mask
    mu = jnp.sum(zm, axis=0, keepdims=True) / B
    var = jnp.sum((z1 - mu) ** 2 * mask, axis=0, keepdims=True) / B
    z1 = (z1 - mu) / jnp.sqrt(var + 1e-5) * g_ref[...] + be_ref[...]
    z1 = jax.nn.relu(z1)
    y = jnp.dot(z1, wr2_ref[...], preferred_element_type=jnp.float32, precision=lax.Precision.HIGHEST) \
        + br2_ref[...]
    y_ref[...] = y[:B, :]


def _tc_head(q, racc, den, Wr1, br1, gamma, beta, Wr2, br2):
    return pl.pallas_call(
        _head_body,
        out_shape=jax.ShapeDtypeStruct((B, 1), jnp.float32),
    )(q, racc, den, Wr1, br1.reshape(1, H), gamma.reshape(1, H),
      beta.reshape(1, H), Wr2, br2.reshape(1, 1))


def kernel(x, edge_index, edge_attr, batch, W0, b0, We1, be1, We2, be2,
           Wroot, broot, Wi, bi, Wh, bh, Wli, bli, Wlh, blh, Wr1, br1,
           gamma, beta, Wr2, br2):
    src = edge_index[0].astype(jnp.int32)
    dst = edge_index[1].astype(jnp.int32)
    pad = EPAD - E
    src2 = jnp.concatenate([src, jnp.zeros((pad,), jnp.int32)]
                           ).reshape(EPAD // IB, IB)
    dst2 = jnp.concatenate([dst, jnp.zeros((pad,), jnp.int32)]
                           ).reshape(EPAD // IB, IB)
    eat = jnp.concatenate(
        [edge_attr.T, jnp.zeros((DE, EPAD - E), jnp.float32)], axis=1)
    ii = jnp.arange(H * H)
    Rm = jnp.where(ii[None, :] // H == jnp.arange(H)[:, None], 1.0, 0.0)
    Sm = jnp.where(ii[:, None] % H == jnp.arange(H)[None, :], 1.0, 0.0)
    zNW = jnp.zeros((ACCROWS, W), jnp.float32)
    batch2 = batch.astype(jnp.int32).reshape(N, 1)

    out = _tc_init(x, W0, b0)
    h = out
    for step in range(MP_STEPS):
        osrc = _sc_gather(out, src2)
        msg = _tc_msg(eat, osrc, We1, be1, We2, be2, Rm, Sm)
        p = _sc_scatter(msg, dst2, zNW)
        out = _tc_update(out, h, p, Wroot, broot, Wi, bi, Wh, bh)
        h = out

    zb = jnp.zeros((BP, H), jnp.float32)
    q = zb
    racc = zb
    den = jnp.ones((BP, 1), jnp.float32)
    hl = zb
    cl = zb
    for step in range(S2S_STEPS):
        hl, cl = _tc_lstm(q, racc, den, hl, cl, Wli, bli, Wlh, blh)
        q = hl
        e, gmax = _tc_passA(out, q, batch2)
        den, racc = _tc_passB(e, gmax, out, batch2)
    y = _tc_head(q, racc, den, Wr1, br1, gamma, beta, Wr2, br2)
    return y.reshape(-1)


# speedup vs baseline: 2.5493x; 2.5493x over previous
import functools

import jax
import jax.numpy as jnp
from jax import lax
from jax.experimental import pallas as pl
from jax.experimental.pallas import tpu as pltpu
from jax.experimental.pallas import tpu_sc as plsc

N = 10000
E = 160000
DF = 128
DE = 16
H = 32
B = 500
BP = 512
MP_STEPS = 3
S2S_STEPS = 3

NC = 2
NS = 16
NW = NC * NS
EPAD = 163840
CHUNK = 512
GC = 256
NCHUNK = EPAD // (NW * GC)
IB = 128
JPER = GC // IB
W = 128
HALF = N // 2
ACCROWS = 5120
TPT = EPAD // NS
SCCH = TPT // GC

@functools.cache
def _sc_kernels():
    mesh = plsc.VectorSubcoreMesh(core_axis_name="c", subcore_axis_name="s",
                                  num_cores=NC, num_subcores=NS)

    @functools.partial(
        pl.kernel,
        out_type=jax.ShapeDtypeStruct((EPAD, W), jnp.float32),
        mesh=mesh,
        scratch_types=[
            pltpu.VMEM((JPER, IB), jnp.int32),
            pltpu.VMEM((JPER, IB), jnp.int32),
            pltpu.VMEM((GC, W), jnp.float32),
            pltpu.VMEM((GC, W), jnp.float32),
            pltpu.SemaphoreType.DMA,
            pltpu.SemaphoreType.DMA,
        ],
    )
    def sc_gather(table_hbm, idx2_hbm, out_hbm, ix0, ix1, rb0, rb1,
                  sem_g, sem_wb):
        ixb = (ix0, ix1)
        rbb = (rb0, rb1)
        wid = lax.axis_index("s") * NC + lax.axis_index("c")
        wbase = pl.multiple_of(wid * (NCHUNK * GC), GC)
        wrow = pl.multiple_of(wid * (NCHUNK * JPER), JPER)
        pltpu.sync_copy(idx2_hbm.at[pl.ds(wrow, JPER)], ixb[0])
        wb = [None] * NCHUNK
        for c in range(NCHUNK):
            b = c % 2
            if c >= 2:
                wb[c - 2].wait()
            g = [pltpu.async_copy(table_hbm.at[ixb[b].at[j]],
                                  rbb[b].at[pl.ds(j * IB, IB)], sem_g)
                 for j in range(JPER)]
            if c + 1 < NCHUNK:
                pltpu.sync_copy(
                    idx2_hbm.at[pl.ds(wrow + (c + 1) * JPER, JPER)],
                    ixb[(c + 1) % 2])
            for d in g:
                d.wait()
            wb[c] = pltpu.async_copy(
                rbb[b], out_hbm.at[pl.ds(wbase + c * GC, GC)], sem_wb)
        wb[NCHUNK - 2].wait()
        wb[NCHUNK - 1].wait()

    @functools.partial(
        pl.kernel,
        out_type=jax.ShapeDtypeStruct((NC, ACCROWS, W), jnp.float32),
        mesh=mesh,
        scratch_types=[
            pltpu.VMEM((JPER, IB), jnp.int32),
            pltpu.VMEM((JPER, IB), jnp.int32),
            pltpu.VMEM((GC, W), jnp.float32),
            pltpu.VMEM((GC, W), jnp.float32),
            pltpu.VMEM_SHARED((ACCROWS, W), jnp.float32),
            pltpu.SemaphoreType.DMA,
            pltpu.SemaphoreType.DMA,
            pltpu.SemaphoreType.DMA,
        ],
    )
    def sc_scatter(msg_hbm, dst2_hbm, z_hbm, out_hbm, ix0, ix1, bf0, bf1,
                   acc_s, sem_ix, sem_in, sem_sc):
        ixb = (ix0, ix1)
        bfb = (bf0, bf1)
        cid = lax.axis_index("c")
        sid = lax.axis_index("s")

        @pl.when(sid == 0)
        def _init():
            pltpu.sync_copy(z_hbm, acc_s)

        nbase = cid * HALF
        tbase = pl.multiple_of(sid * TPT, GC)
        trow = pl.multiple_of(sid * (TPT // IB), JPER)
        plsc.subcore_barrier()

        def stage(c):
            b = c % 2
            di = pltpu.async_copy(
                dst2_hbm.at[pl.ds(trow + c * JPER, JPER)], ixb[b], sem_ix)
            dm = pltpu.async_copy(
                msg_hbm.at[pl.ds(tbase + c * GC, GC)], bfb[b], sem_in)
            return di, dm

        st = {0: stage(0), 1: stage(1)}
        sc = {}
        for c in range(SCCH):
            b = c % 2
            di, dm = st[c]
            di.wait()
            dm.wait()
            for r in range(JPER):
                for kk in range(IB // 16):
                    v = ixb[b][r, pl.ds(kk * 16, 16)] - nbase
                    ok = (v >= 0) & (v < HALF)
                    ixb[b][r, pl.ds(kk * 16, 16)] = jnp.where(ok, v, HALF)
            sc[c] = [pltpu.async_copy(bfb[b].at[pl.ds(r * IB, IB)],
                                      acc_s.at[ixb[b].at[r]], sem_sc,
                                      add=True)
                     for r in range(JPER)]
            if c + 2 < SCCH:
                for d in sc[c]:
                    d.wait()
                st[c + 2] = stage(c + 2)
        for d in sc[SCCH - 2] + sc[SCCH - 1]:
            d.wait()
        plsc.subcore_barrier()

        @pl.when(sid == 0)
        def _flush():
            pltpu.sync_copy(acc_s, out_hbm.at[cid])

    return sc_gather, sc_scatter


def _sc_gather(table, idx2):
    return _sc_kernels()[0](table, idx2)


def _sc_scatter(msg, dst2, z):
    return _sc_kernels()[1](msg, dst2, z)


def _init_body(x_ref, w_ref, b_ref, o_ref):
    o = jax.nn.relu(
        jnp.dot(x_ref[...], w_ref[...], preferred_element_type=jnp.float32, precision=lax.Precision.HIGHEST)
        + b_ref[...])
    o_ref[...] = jnp.concatenate(
        [o, jnp.zeros((N, W - H), jnp.float32)], axis=1)


def _tc_init(x, W0, b0):
    return pl.pallas_call(
        _init_body,
        out_shape=jax.ShapeDtypeStruct((N, W), jnp.float32),
    )(x, W0, b0.reshape(1, H))


def _msg_body(eat_ref, os_ref, w1t_ref, b1c_ref, w2t_ref, b2c_ref, o_ref):
    i = pl.program_id(0)
    hidT = jax.nn.relu(
        jnp.dot(w1t_ref[...], eat_ref[...],
                preferred_element_type=jnp.float32) + b1c_ref[...])
    ewT = jnp.dot(w2t_ref[...], hidT,
                  preferred_element_type=jnp.float32) + b2c_ref[...]
    osT = jnp.transpose(os_ref[:, :H])
    osx = jnp.broadcast_to(osT[:, None, :], (H, H, CHUNK)).reshape(
        H * H, CHUNK)
    msgT = (ewT * osx).reshape(H, H, CHUNK).sum(axis=0)
    msg = jnp.transpose(msgT)
    gid = i * CHUNK + lax.broadcasted_iota(jnp.int32, (CHUNK, 1), 0)
    real = jnp.where(gid < E, 1.0, 0.0)
    msg = jnp.where(gid < E, msg, 0.0)
    o_ref[...] = jnp.concatenate(
        [msg, real, jnp.zeros((CHUNK, W - H - 1), jnp.float32)], axis=1)


def _tc_msg(eat, osrc, W1T, b1c, W2T, b2c):
    nblk = EPAD // CHUNK
    return pl.pallas_call(
        _msg_body,
        grid=(nblk,),
        in_specs=[
            pl.BlockSpec((DE, CHUNK), lambda i: (0, i)),
            pl.BlockSpec((CHUNK, W), lambda i: (i, 0)),
            pl.BlockSpec((2 * H, DE), lambda i: (0, 0)),
            pl.BlockSpec((2 * H, 1), lambda i: (0, 0)),
            pl.BlockSpec((H * H, 2 * H), lambda i: (0, 0)),
            pl.BlockSpec((H * H, 1), lambda i: (0, 0)),
        ],
        out_specs=pl.BlockSpec((CHUNK, W), lambda i: (i, 0)),
        out_shape=jax.ShapeDtypeStruct((EPAD, W), jnp.float32),
    )(eat, osrc, W1T, b1c, W2T, b2c)


def _update_body(out_ref, h_ref, p_ref, wroot_ref, broot_ref,
                 wi_ref, bi_ref, wh_ref, bh_ref, o_ref):
    out = out_ref[:, :H]
    h = h_ref[:, :H]
    psum = jnp.concatenate(
        [p_ref[0, :HALF, :], p_ref[1, :HALF, :]], axis=0)
    deg = jnp.maximum(psum[:, H:H + 1], 1.0)
    aggr = psum[:, :H] / deg
    m = jax.nn.relu(
        jnp.dot(out, wroot_ref[...], preferred_element_type=jnp.float32, precision=lax.Precision.HIGHEST)
        + broot_ref[...] + aggr)
    gi = jnp.dot(m, wi_ref[...], preferred_element_type=jnp.float32, precision=lax.Precision.HIGHEST) \
        + bi_ref[...]
    gh = jnp.dot(h, wh_ref[...], preferred_element_type=jnp.float32, precision=lax.Precision.HIGHEST) \
        + bh_ref[...]
    ir, iz, inn = gi[:, :H], gi[:, H:2 * H], gi[:, 2 * H:]
    hr, hz, hn = gh[:, :H], gh[:, H:2 * H], gh[:, 2 * H:]
    r = jax.nn.sigmoid(ir + hr)
    z = jax.nn.sigmoid(iz + hz)
    n = jnp.tanh(inn + r * hn)
    hnew = (1.0 - z) * n + z * h
    o_ref[...] = jnp.concatenate(
        [hnew, jnp.zeros((N, W - H), jnp.float32)], axis=1)


def _tc_update(out, h, p, Wroot, broot, Wi, bi, Wh, bh):
    return pl.pallas_call(
        _update_body,
        out_shape=jax.ShapeDtypeStruct((N, W), jnp.float32),
    )(out, h, p, Wroot, broot.reshape(1, H), Wi, bi.reshape(1, 3 * H),
      Wh, bh.reshape(1, 3 * H))


def _lstm_body(qp_ref, racc_ref, den_ref, hl_ref, cl_ref,
               wli_ref, bli_ref, wlh_ref, blh_ref, hlo_ref, clo_ref):
    r = racc_ref[...] / jnp.maximum(den_ref[...], 1e-16)
    q_star = jnp.concatenate([qp_ref[...], r], axis=1)
    g = jnp.dot(q_star, wli_ref[...], preferred_element_type=jnp.float32, precision=lax.Precision.HIGHEST) \
        + bli_ref[...] \
        + jnp.dot(hl_ref[...], wlh_ref[...],
                  preferred_element_type=jnp.float32, precision=lax.Precision.HIGHEST) + blh_ref[...]
    ig = jax.nn.sigmoid(g[:, :H])
    fg = jax.nn.sigmoid(g[:, H:2 * H])
    gg = jnp.tanh(g[:, 2 * H:3 * H])
    og = jax.nn.sigmoid(g[:, 3 * H:])
    cl = fg * cl_ref[...] + ig * gg
    clo_ref[...] = cl
    hlo_ref[...] = og * jnp.tanh(cl)


def _tc_lstm(qp, racc, den, hl, cl, Wli, bli, Wlh, blh):
    return pl.pallas_call(
        _lstm_body,
        out_shape=(
            jax.ShapeDtypeStruct((BP, H), jnp.float32),
            jax.ShapeDtypeStruct((BP, H), jnp.float32),
        ),
    )(qp, racc, den, hl, cl, Wli, bli.reshape(1, 4 * H),
      Wlh, blh.reshape(1, 4 * H))


NB = 2000
NBLK = N // NB


def _passA_body(out_ref, q_ref, b_ref, e_ref, gmax_ref):
    i = pl.program_id(0)
    oh = jnp.where(
        lax.broadcasted_iota(jnp.int32, (NB, BP), 1) == b_ref[...],
        1.0, 0.0)
    qn = jnp.dot(oh, q_ref[...], preferred_element_type=jnp.float32, precision=lax.Precision.HIGHEST)
    e = jnp.sum(out_ref[:, :H] * qn, axis=1, keepdims=True)
    e_ref[...] = e

    @pl.when(i == 0)
    def _():
        gmax_ref[...] = jnp.full((1, 1), -3e38, jnp.float32)

    gmax_ref[...] = jnp.maximum(gmax_ref[...], jnp.max(e))


def _tc_passA(out, q, batch2):
    return pl.pallas_call(
        _passA_body,
        grid=(NBLK,),
        in_specs=[
            pl.BlockSpec((NB, W), lambda i: (i, 0)),
            pl.BlockSpec((BP, H), lambda i: (0, 0)),
            pl.BlockSpec((NB, 1), lambda i: (i, 0)),
        ],
        out_specs=(
            pl.BlockSpec((NB, 1), lambda i: (i, 0)),
            pl.BlockSpec((1, 1), lambda i: (0, 0)),
        ),
        out_shape=(
            jax.ShapeDtypeStruct((N, 1), jnp.float32),
            jax.ShapeDtypeStruct((1, 1), jnp.float32),
        ),
    )(out, q, batch2)


def _passB_body(e_ref, gmax_ref, out_ref, b_ref, den_ref, racc_ref):
    i = pl.program_id(0)
    a = jnp.exp(e_ref[...] - gmax_ref[0, 0])
    oh = jnp.where(
        lax.broadcasted_iota(jnp.int32, (NB, BP), 1) == b_ref[...],
        1.0, 0.0)
    dn = lax.dot_general(oh, a, (((0,), (0,)), ((), ())),
                         preferred_element_type=jnp.float32, precision=lax.Precision.HIGHEST)
    rc = lax.dot_general(oh, a * out_ref[:, :H], (((0,), (0,)), ((), ())),
                         preferred_element_type=jnp.float32, precision=lax.Precision.HIGHEST)

    @pl.when(i == 0)
    def _():
        den_ref[...] = jnp.zeros((BP, 1), jnp.float32)
        racc_ref[...] = jnp.zeros((BP, H), jnp.float32)

    den_ref[...] += dn
    racc_ref[...] += rc


def _tc_passB(e, gmax, out, batch2):
    return pl.pallas_call(
        _passB_body,
        grid=(NBLK,),
        in_specs=[
            pl.BlockSpec((NB, 1), lambda i: (i, 0)),
            pl.BlockSpec((1, 1), lambda i: (0, 0)),
            pl.BlockSpec((NB, W), lambda i: (i, 0)),
            pl.BlockSpec((NB, 1), lambda i: (i, 0)),
        ],
        out_specs=(
            pl.BlockSpec((BP, 1), lambda i: (0, 0)),
            pl.BlockSpec((BP, H), lambda i: (0, 0)),
        ),
        out_shape=(
            jax.ShapeDtypeStruct((BP, 1), jnp.float32),
            jax.ShapeDtypeStruct((BP, H), jnp.float32),
        ),
    )(e, gmax, out, batch2)


def _head_body(q_ref, racc_ref, den_ref, wr1_ref, br1_ref, g_ref, be_ref,
               wr2_ref, br2_ref, y_ref):
    r = racc_ref[...] / jnp.maximum(den_ref[...], 1e-16)
    q_star = jnp.concatenate([q_ref[...], r], axis=1)
    z1 = jnp.dot(q_star, wr1_ref[...], preferred_element_type=jnp.float32, precision=lax.Precision.HIGHEST) \
        + br1_ref[...]
    mask = jnp.where(
        lax.broadcasted_iota(jnp.int32, (BP, 1), 0) < B, 1.0, 0.0)
    zm = z1 * mask
    mu = jnp.sum(zm, axis=0, keepdims=True) / B
    var = jnp.sum((z1 - mu) ** 2 * mask, axis=0, keepdims=True) / B
    z1 = (z1 - mu) / jnp.sqrt(var + 1e-5) * g_ref[...] + be_ref[...]
    z1 = jax.nn.relu(z1)
    y = jnp.dot(z1, wr2_ref[...], preferred_element_type=jnp.float32, precision=lax.Precision.HIGHEST) \
        + br2_ref[...]
    y_ref[...] = y[:B, :]


def _tc_head(q, racc, den, Wr1, br1, gamma, beta, Wr2, br2):
    return pl.pallas_call(
        _head_body,
        out_shape=jax.ShapeDtypeStruct((B, 1), jnp.float32),
    )(q, racc, den, Wr1, br1.reshape(1, H), gamma.reshape(1, H),
      beta.reshape(1, H), Wr2, br2.reshape(1, 1))


def kernel(x, edge_index, edge_attr, batch, W0, b0, We1, be1, We2, be2,
           Wroot, broot, Wi, bi, Wh, bh, Wli, bli, Wlh, blh, Wr1, br1,
           gamma, beta, Wr2, br2):
    src = edge_index[0].astype(jnp.int32)
    dst = edge_index[1].astype(jnp.int32)
    pad = EPAD - E
    src2 = jnp.concatenate([src, jnp.zeros((pad,), jnp.int32)]
                           ).reshape(EPAD // IB, IB)
    dst2 = jnp.concatenate([dst, jnp.zeros((pad,), jnp.int32)]
                           ).reshape(EPAD // IB, IB)
    eat = jnp.concatenate(
        [edge_attr.T, jnp.zeros((DE, EPAD - E), jnp.float32)], axis=1)
    W1T = We1.T
    b1c = be1.reshape(2 * H, 1)
    W2T = We2.T
    b2c = be2.reshape(H * H, 1)
    zNW = jnp.zeros((ACCROWS, W), jnp.float32)
    batch2 = batch.astype(jnp.int32).reshape(N, 1)

    out = _tc_init(x, W0, b0)
    h = out
    for step in range(MP_STEPS):
        osrc = _sc_gather(out, src2)
        msg = _tc_msg(eat, osrc, W1T, b1c, W2T, b2c)
        p = _sc_scatter(msg, dst2, zNW)
        out = _tc_update(out, h, p, Wroot, broot, Wi, bi, Wh, bh)
        h = out

    zb = jnp.zeros((BP, H), jnp.float32)
    q = zb
    racc = zb
    den = jnp.ones((BP, 1), jnp.float32)
    hl = zb
    cl = zb
    for step in range(S2S_STEPS):
        hl, cl = _tc_lstm(q, racc, den, hl, cl, Wli, bli, Wlh, blh)
        q = hl
        e, gmax = _tc_passA(out, q, batch2)
        den, racc = _tc_passB(e, gmax, out, batch2)
    y = _tc_head(q, racc, den, Wr1, br1, gamma, beta, Wr2, br2)
    return y.reshape(-1)
